# baseline (speedup 1.0000x reference)
import functools

import jax
import jax.numpy as jnp
from jax import lax
from jax.experimental import pallas as pl
from jax.experimental.pallas import tpu as pltpu
from jax.experimental.pallas import tpu_sc as plsc

N_ROWS = 262144
N_UPD = 16384
D = 64
NC = 2
NS = 16
LANES = 16
CH = 8192
N_CHUNK = N_ROWS // CH
CHUNKS_PER_CORE = N_CHUNK // NC
UPD_PER_TILE = N_UPD // NS
NVEC = UPD_PER_TILE // LANES
GROUP = 128
MAX_G = UPD_PER_TILE // GROUP
ROWS_PER_TILE = CH // NS
LIST_CAP = UPD_PER_TILE + GROUP
N_PAD = 8
POS_BITS = 15
POS_MASK = (1 << POS_BITS) - 1
ROW_MASK = CH - 1


def _sc_body(idx_hbm, a_hbm, b_hbm, out_hbm,
             idx_v, list_f, gidx_l, gidx_p, stage, buf0, buf1,
             lsem, ssem, gsem, asem):
    c = lax.axis_index("c")
    s = lax.axis_index("s")
    tbase = s * UPD_PER_TILE
    lanes = lax.iota(jnp.int32, LANES)
    dummy_vec = ((lanes & 7) << POS_BITS) | (N_UPD + (lanes & 7))
    pltpu.sync_copy(idx_hbm.at[pl.ds(tbase, UPD_PER_TILE)], idx_v)

    bufs = (buf0, buf1)
    own = pl.ds(s * ROWS_PER_TILE, ROWS_PER_TILE)

    def scan_chunk(base):
        def scan(v, cnt):
            iv = idx_v[pl.ds(v * LANES, LANES)]
            m = (iv >= base) & (iv < base + CH)
            packed = ((iv - base) << POS_BITS) | (tbase + v * LANES + lanes)
            p = jnp.where(m, jnp.int32(1), jnp.int32(0))
            for sh in (1, 2, 4, 8):
                moved = p[jnp.maximum(lanes - sh, 0)]
                p = p + jnp.where(lanes >= sh, moved, 0)
            h = p[15]
            src = jnp.zeros((LANES,), jnp.int32)
            for sh in (8, 4, 2, 1):
                t = src + sh
                pv = p[jnp.minimum(t - 1, 15)]
                src = jnp.where(pv < lanes + 1, t, src)
            comp = packed[jnp.minimum(src, 15)]
            comp = jnp.where(lanes < h, comp, dummy_vec)
            list_f[pl.ds(cnt, LANES)] = comp
            return cnt + h
        cnt = lax.fori_loop(0, NVEC, scan, jnp.int32(0))
        for t in range(MAX_G):
            list_f[pl.ds(cnt + t * LANES, LANES)] = dummy_vec
        return cnt

    def apply_groups(ng, buf):
        def group_body(g, _):
            def cp_inner(kk, _):
                v = list_f[pl.ds(g * GROUP + kk * LANES, LANES)]
                gidx_l[pl.ds(kk * LANES, LANES)] = (v >> POS_BITS) & ROW_MASK
                gidx_p[pl.ds(kk * LANES, LANES)] = jnp.minimum(
                    v & POS_MASK, N_UPD + N_PAD - 1)
                return 0
            lax.fori_loop(0, GROUP // LANES, cp_inner, 0)
            pltpu.async_copy(b_hbm.at[gidx_p], stage, gsem).wait()
            pltpu.async_copy(stage, buf.at[gidx_l], asem, add=True).wait()
            return 0
        lax.fori_loop(0, ng, group_body, 0)

    load_d = [None] * CHUNKS_PER_CORE
    store_d = [None] * CHUNKS_PER_CORE

    def issue_load(k):
        base = (c * CHUNKS_PER_CORE + k) * CH
        load_d[k] = pltpu.async_copy(
            a_hbm.at[pl.ds(base + s * ROWS_PER_TILE, ROWS_PER_TILE)],
            bufs[k & 1].at[own], lsem)

    issue_load(0)
    for k in range(CHUNKS_PER_CORE):
        buf = bufs[k & 1]
        base = (c * CHUNKS_PER_CORE + k) * CH
        cnt = scan_chunk(base)
        ng = (cnt + GROUP - 1) // GROUP
        load_d[k].wait()
        plsc.subcore_barrier()
        if k + 1 < CHUNKS_PER_CORE:
            if k >= 1:
                store_d[k - 1].wait()
            issue_load(k + 1)
        apply_groups(ng, buf)
        plsc.subcore_barrier()
        store_d[k] = pltpu.async_copy(
            buf.at[own],
            out_hbm.at[pl.ds(base + s * ROWS_PER_TILE, ROWS_PER_TILE)], ssem)
    store_d[CHUNKS_PER_CORE - 2].wait()
    store_d[CHUNKS_PER_CORE - 1].wait()


_scatter_add = functools.partial(
    pl.kernel,
    out_type=jax.ShapeDtypeStruct((N_ROWS, D), jnp.float32),
    mesh=plsc.VectorSubcoreMesh(core_axis_name="c", subcore_axis_name="s"),
    compiler_params=pltpu.CompilerParams(use_tc_tiling_on_sc=False),
    scratch_types=[
        pltpu.VMEM((UPD_PER_TILE,), jnp.int32),
        pltpu.VMEM((LIST_CAP,), jnp.int32),
        pltpu.VMEM((GROUP,), jnp.int32),
        pltpu.VMEM((GROUP,), jnp.int32),
        pltpu.VMEM((GROUP, D), jnp.float32),
        pltpu.VMEM_SHARED((CH, D), jnp.float32),
        pltpu.VMEM_SHARED((CH, D), jnp.float32),
        pltpu.SemaphoreType.DMA,
        pltpu.SemaphoreType.DMA,
        pltpu.SemaphoreType.DMA,
        pltpu.SemaphoreType.DMA,
    ],
)(_sc_body)


def kernel(index, A, B):
    b_ext = jnp.concatenate([B, jnp.zeros((N_PAD, D), B.dtype)], axis=0)
    return _scatter_add(index.astype(jnp.int32), A, b_ext)
